# baseline (speedup 1.0000x reference)
import jax
import jax.numpy as jnp
from jax.experimental import pallas as pl
from jax.experimental.pallas import tpu as pltpu

_B = 512 * 1024
_K = 6
_R = 3


def _make_body(n):
    nc = pl.cdiv(n, _B)
    tail = n - (nc - 1) * _B

    def body(v_ref, o_ref, *scratch):
        bufs = scratch[:_K]
        tail_buf = scratch[_K]
        rsems = scratch[_K + 1:2 * _K + 1]
        wsems = scratch[2 * _K + 1:3 * _K + 1]
        tsems = scratch[3 * _K + 1:]

        def rd(i):
            if i == nc - 1:
                return pltpu.make_async_copy(
                    v_ref.at[pl.ds(i * _B, tail)], tail_buf, tsems[0])
            return pltpu.make_async_copy(
                v_ref.at[pl.ds(i * _B, _B)], bufs[i % _K], rsems[i % _K])

        def wr(i):
            if i == nc - 1:
                return pltpu.make_async_copy(
                    tail_buf, o_ref.at[pl.ds(i * _B, tail)], tsems[1])
            return pltpu.make_async_copy(
                bufs[i % _K], o_ref.at[pl.ds(i * _B, _B)], wsems[i % _K])

        for i in range(min(_R, nc)):
            rd(i).start()
        for i in range(nc):
            if 0 <= i - (_K - _R) and i + _R < nc:
                wr(i - (_K - _R)).wait()
            rd(i).wait()
            wr(i).start()
            if i + _R < nc:
                rd(i + _R).start()
        waited = {i - (_K - _R) for i in range(nc)
                  if 0 <= i - (_K - _R) and i + _R < nc}
        for i in range(nc):
            if i not in waited:
                wr(i).wait()

    return body


def kernel(values, indices):
    n = values.shape[0]
    nc = pl.cdiv(n, _B)
    tail = n - (nc - 1) * _B
    return pl.pallas_call(
        _make_body(n),
        in_specs=[pl.BlockSpec(memory_space=pltpu.MemorySpace.HBM)],
        out_specs=pl.BlockSpec(memory_space=pltpu.MemorySpace.HBM),
        out_shape=jax.ShapeDtypeStruct(values.shape, values.dtype),
        scratch_shapes=(
            [pltpu.VMEM((_B,), jnp.float32)] * _K
            + [pltpu.VMEM((tail,), jnp.float32)]
            + [pltpu.SemaphoreType.DMA] * (2 * _K + 2)
        ),
    )(values)


# speedup vs baseline: 1.0131x; 1.0131x over previous
import jax
import jax.numpy as jnp
from jax.experimental import pallas as pl
from jax.experimental.pallas import tpu as pltpu

_B = 1024 * 1024
_K = 5
_R = 2


def _make_body(n):
    nc = pl.cdiv(n, _B)
    tail = n - (nc - 1) * _B

    def body(v_ref, o_ref, *scratch):
        bufs = scratch[:_K]
        tail_buf = scratch[_K]
        rsems = scratch[_K + 1:2 * _K + 1]
        wsems = scratch[2 * _K + 1:3 * _K + 1]
        tsems = scratch[3 * _K + 1:]

        def rd(i):
            if i == nc - 1:
                return pltpu.make_async_copy(
                    v_ref.at[pl.ds(i * _B, tail)], tail_buf, tsems[0])
            return pltpu.make_async_copy(
                v_ref.at[pl.ds(i * _B, _B)], bufs[i % _K], rsems[i % _K])

        def wr(i):
            if i == nc - 1:
                return pltpu.make_async_copy(
                    tail_buf, o_ref.at[pl.ds(i * _B, tail)], tsems[1])
            return pltpu.make_async_copy(
                bufs[i % _K], o_ref.at[pl.ds(i * _B, _B)], wsems[i % _K])

        for i in range(min(_R, nc)):
            rd(i).start()
        for i in range(nc):
            if 0 <= i - (_K - _R) and i + _R < nc:
                wr(i - (_K - _R)).wait()
            rd(i).wait()
            wr(i).start()
            if i + _R < nc:
                rd(i + _R).start()
        waited = {i - (_K - _R) for i in range(nc)
                  if 0 <= i - (_K - _R) and i + _R < nc}
        for i in range(nc):
            if i not in waited:
                wr(i).wait()

    return body


def kernel(values, indices):
    n = values.shape[0]
    nc = pl.cdiv(n, _B)
    tail = n - (nc - 1) * _B
    return pl.pallas_call(
        _make_body(n),
        in_specs=[pl.BlockSpec(memory_space=pltpu.MemorySpace.HBM)],
        out_specs=pl.BlockSpec(memory_space=pltpu.MemorySpace.HBM),
        out_shape=jax.ShapeDtypeStruct(values.shape, values.dtype),
        scratch_shapes=(
            [pltpu.VMEM((_B,), jnp.float32)] * _K
            + [pltpu.VMEM((tail,), jnp.float32)]
            + [pltpu.SemaphoreType.DMA] * (2 * _K + 2)
        ),
    )(values)


# speedup vs baseline: 1.0901x; 1.0760x over previous
import jax
import jax.numpy as jnp
from jax.experimental import pallas as pl

_BLOCK = 1536 * 1024


def _copy_block(v_ref, o_ref):
    o_ref[...] = v_ref[...]


def kernel(values, indices):
    n = values.shape[0]
    grid = (pl.cdiv(n, _BLOCK),)
    return pl.pallas_call(
        _copy_block,
        grid=grid,
        in_specs=[pl.BlockSpec((_BLOCK,), lambda i: (i,))],
        out_specs=pl.BlockSpec((_BLOCK,), lambda i: (i,)),
        out_shape=jax.ShapeDtypeStruct(values.shape, values.dtype),
    )(values)
